# baseline (speedup 1.0000x reference)
import functools

import jax
import jax.numpy as jnp
from jax import lax
from jax.experimental import pallas as pl
from jax.experimental.pallas import tpu as pltpu
from jax.experimental.pallas import tpu_sc as plsc

NC, NS = 2, 16
NW = NC * NS
CB = 128
NBUF = 8
LAG = 4


@functools.partial(jax.jit, static_argnums=(2, 3))
def _gather(idx_flat, table, nch, d):
    rpw = nch * CB
    mesh = plsc.VectorSubcoreMesh(core_axis_name="c", subcore_axis_name="s")

    @functools.partial(
        pl.kernel,
        out_type=jax.ShapeDtypeStruct((NW * rpw, d), jnp.float32),
        mesh=mesh,
        scratch_types=[
            pltpu.VMEM((nch * CB,), jnp.int32),
            pltpu.VMEM((NBUF, CB, d), jnp.float32),
            pltpu.SemaphoreType.DMA((NBUF,)),
            pltpu.SemaphoreType.DMA((NBUF,)),
        ],
        compiler_params=pltpu.CompilerParams(use_tc_tiling_on_sc=False),
    )
    def k(idx_hbm, table_hbm, out_hbm, idx_v, rows_v, gsem, osem):
        wid = lax.axis_index("s") * NC + lax.axis_index("c")
        base = wid * rpw
        pltpu.sync_copy(idx_hbm.at[pl.ds(base, rpw)], idx_v)

        def start_gather(j, slot):
            pltpu.async_copy(
                table_hbm.at[idx_v.at[pl.ds(j * CB, CB)]], rows_v.at[slot],
                gsem.at[slot])

        def drain_chunk(jd, slot):
            pltpu.make_async_copy(
                table_hbm.at[idx_v.at[pl.ds(jd * CB, CB)]], rows_v.at[slot],
                gsem.at[slot]
            ).wait()
            pltpu.async_copy(
                rows_v.at[slot], out_hbm.at[pl.ds(base + jd * CB, CB)],
                osem.at[slot])

        def wait_out(jd, slot):
            pltpu.make_async_copy(
                rows_v.at[slot], out_hbm.at[pl.ds(base + jd * CB, CB)],
                osem.at[slot]
            ).wait()

        for j in range(NBUF):
            start_gather(j, j)
            if j >= LAG:
                drain_chunk(j - LAG, j - LAG)

        def body(j, carry):
            slot = jnp.bitwise_and(j, NBUF - 1)
            jd = j - NBUF
            wait_out(jd, slot)
            start_gather(j, slot)
            jw = j - LAG
            drain_chunk(jw, jnp.bitwise_and(jw, NBUF - 1))
            return carry

        lax.fori_loop(NBUF, nch, body, 0)

        for jd in range(nch - LAG, nch):
            drain_chunk(jd, jd % NBUF)
        for jd in range(nch - NBUF, nch):
            wait_out(jd, jd % NBUF)

    return k(idx_flat, table)


def _relayout_body(i_ref, o_ref):
    x = i_ref[...]
    a = x[:, :64]
    b = x[:, 64:]
    y = jnp.concatenate([a[:, None, :], b[:, None, :]], axis=1)
    o_ref[...] = y.reshape(o_ref.shape)


@functools.partial(jax.jit, static_argnums=(1, 2, 3))
def _relayout(rows128, b, s, d):
    nb = 64
    rows_per_step = nb * s * d // 128
    return pl.pallas_call(
        _relayout_body,
        grid=(b // nb,),
        in_specs=[pl.BlockSpec((rows_per_step, 128), lambda i: (i, 0))],
        out_specs=pl.BlockSpec((nb, s, d), lambda i: (i, 0, 0)),
        out_shape=jax.ShapeDtypeStruct((b, s, d), jnp.float32),
    )(rows128)


def kernel(idx, weight):
    b, s = idx.shape
    d = weight.shape[-1]
    nch = (b * s) // (NW * CB)
    idx_flat = idx.reshape(-1).astype(jnp.int32)
    out = _gather(idx_flat, weight, nch, d)
    return _relayout(out.reshape(b * s * d // 128, 128), b, s, d)
